# baseline (speedup 1.0000x reference)
import functools

import jax
import jax.numpy as jnp
from jax.experimental import pallas as pl

_B = 4096
_D = 64
_TM = 512
_NEG_INF = float("-inf")
_TINY = float(jnp.finfo(jnp.float32).tiny)


def _threefry_bits(kd0, kd1, idx, offset=0):
    ks2 = (0x1BD11BDA ^ kd0 ^ kd1) & 0xFFFFFFFF
    rots = ((13, 15, 26, 6), (17, 29, 16, 24))
    keys = (kd0, kd1, ks2)
    x0 = jnp.full_like(idx, jnp.uint32(kd0))
    x1 = idx + (jnp.uint32(kd1) + jnp.uint32(offset))
    for i in range(5):
        for rot in rots[i % 2]:
            x0 = x0 + x1
            x1 = (x1 << rot) | (x1 >> (32 - rot))
            x1 = x1 ^ x0
        x0 = x0 + jnp.uint32(keys[(i + 1) % 3])
        x1 = x1 + jnp.uint32((keys[(i + 2) % 3] + i + 1) & 0xFFFFFFFF)
    return x0 ^ x1


def _gumbel_from_bits(bits):
    fb = (bits >> jnp.uint32(9)) | jnp.uint32(0x3F800000)
    floats = jax.lax.bitcast_convert_type(fb, jnp.float32) - jnp.float32(1.0)
    u = jnp.maximum(jnp.float32(_TINY), floats)
    return -jnp.log(-jnp.log(u))


_KD1 = (1832780943, 270669613)
_KD2 = (64467757, 2916123636)


def _body(xr_ref, xf_ref, nr_ref, nf_ref, g1_ref,
          loss_ref, act_ref, acc_ref):
    t = pl.program_id(0)
    row0 = t * _TM

    xr = xr_ref[...]
    xf = xf_ref[...]
    r = jax.lax.dot_general(
        xr, xf, (((1,), (1,)), ((), ())),
        preferred_element_type=jnp.float32)
    d = nr_ref[...] + nf_ref[...] - 2.0 * r
    d = jnp.maximum(d, 1e-08)

    rows = row0 + jax.lax.broadcasted_iota(jnp.int32, (_TM, _B), 0)
    cols = jax.lax.broadcasted_iota(jnp.int32, (_TM, _B), 1)
    in_class = (rows // 16) == (cols // 16)

    dist_pos_full = jnp.where(in_class & (rows != cols), d, 0.0)
    dist_neg = jnp.where(in_class, 1e25, d)

    maxp = jnp.max(dist_pos_full, axis=1, keepdims=True)
    minn = jnp.min(dist_neg, axis=1, keepdims=True)
    acc_flags = (jnp.sqrt(maxp) < jnp.sqrt(minn)).astype(jnp.float32)

    r_in = jax.lax.dot_general(
        xr, xr, (((1,), (1,)), ((), ())),
        preferred_element_type=jnp.float32)
    d_in = nr_ref[...] + nf_ref[:, pl.ds(row0, _TM)] - 2.0 * r_in
    d_in = jnp.maximum(d_in, 1e-08)
    li = jax.lax.broadcasted_iota(jnp.int32, (_TM, _TM), 0)
    lj = jax.lax.broadcasted_iota(jnp.int32, (_TM, _TM), 1)
    pos_ok = ((li // 16) == (lj // 16)) & (li != lj)
    dp_in = jnp.where(pos_ok, d_in, 0.0)
    plog = jnp.where(dp_in > 0.0, jnp.log(jnp.maximum(dp_in, 1e-30)), _NEG_INF)
    g1t = jnp.tile(g1_ref[...], (1, _TM // 16))
    pv = plog + g1t
    m1 = jnp.max(pv, axis=1, keepdims=True)
    idx1 = jnp.min(jnp.where(pv == m1, lj, _B), axis=1, keepdims=True)
    pscore = jnp.sum(jnp.where(lj == idx1, dp_in, 0.0), axis=1, keepdims=True)

    lflat = (jax.lax.broadcasted_iota(jnp.uint32, (_TM, _B), 0) << 12) \
        | jax.lax.broadcasted_iota(jnp.uint32, (_TM, _B), 1)
    g2 = _gumbel_from_bits(
        _threefry_bits(_KD2[0], _KD2[1], lflat,
                       jnp.uint32(t) * jnp.uint32(_TM * _B)))
    candm = dist_neg < (pscore + 0.0001)
    nv = jnp.where(candm, -dist_neg, _NEG_INF) + g2
    m2 = jnp.max(nv, axis=1, keepdims=True)
    idx2 = jnp.min(jnp.where(nv == m2, cols, _B), axis=1, keepdims=True)
    has = m2 > _NEG_INF
    negid = jnp.where(has, idx2, 0)
    sneg = jnp.sum(jnp.where(cols == negid, dist_neg, 0.0), axis=1, keepdims=True)

    score_pos = jnp.sqrt(pscore)
    score_neg = jnp.sqrt(sneg)
    diff = (0.0001 + score_pos) - score_neg
    loss_ref[...] = jnp.logaddexp(diff, 0.0)

    @pl.when(t == 0)
    def _init():
        act_ref[...] = jnp.zeros((1, 1), jnp.float32)
        acc_ref[...] = jnp.zeros((1, 1), jnp.float32)

    act_ref[...] += jnp.sum((diff > 0.0).astype(jnp.float32), keepdims=True)
    acc_ref[...] += jnp.sum(acc_flags, keepdims=True)


@functools.partial(jax.jit, static_argnames=("interpret",))
def _run(features, interpret=False):
    feat_norm = jnp.sum(features * features, axis=1, keepdims=True)
    i = jnp.arange(_B, dtype=jnp.uint32)[:, None]
    kk = jnp.arange(16, dtype=jnp.uint32)[None, :]
    flat = i * jnp.uint32(_B) + (i // jnp.uint32(16)) * jnp.uint32(16) + kk
    g1d = _gumbel_from_bits(_threefry_bits(_KD1[0], _KD1[1], flat))

    loss, act, acc = pl.pallas_call(
        _body,
        grid=(_B // _TM,),
        in_specs=[
            pl.BlockSpec((_TM, _D), lambda t: (t, 0)),
            pl.BlockSpec((_B, _D), lambda t: (0, 0)),
            pl.BlockSpec((_TM, 1), lambda t: (t, 0)),
            pl.BlockSpec((1, _B), lambda t: (0, 0)),
            pl.BlockSpec((_TM, 16), lambda t: (t, 0)),
        ],
        out_specs=[
            pl.BlockSpec((_TM, 1), lambda t: (t, 0)),
            pl.BlockSpec((1, 1), lambda t: (0, 0)),
            pl.BlockSpec((1, 1), lambda t: (0, 0)),
        ],
        out_shape=[
            jax.ShapeDtypeStruct((_B, 1), jnp.float32),
            jax.ShapeDtypeStruct((1, 1), jnp.float32),
            jax.ShapeDtypeStruct((1, 1), jnp.float32),
        ],
        interpret=interpret,
    )(features, features, feat_norm, feat_norm.T, g1d)

    avg_active = act[0, 0] / _B
    accuracy = 100.0 * acc[0, 0] / _B
    return loss, avg_active, accuracy


def kernel(features):
    return _run(features)


# speedup vs baseline: 1.2318x; 1.2318x over previous
import functools

import jax
import jax.numpy as jnp
from jax.experimental import pallas as pl

_B = 4096
_D = 64
_TM = 128
_NEG_INF = float("-inf")
_TINY = float(jnp.finfo(jnp.float32).tiny)


def _threefry_bits(kd0, kd1, idx, offset=0):
    ks2 = (0x1BD11BDA ^ kd0 ^ kd1) & 0xFFFFFFFF
    rots = ((13, 15, 26, 6), (17, 29, 16, 24))
    keys = (kd0, kd1, ks2)
    x0 = jnp.full_like(idx, jnp.uint32(kd0))
    x1 = idx + (jnp.uint32(kd1) + jnp.uint32(offset))
    for i in range(5):
        for rot in rots[i % 2]:
            x0 = x0 + x1
            x1 = (x1 << rot) | (x1 >> (32 - rot))
            x1 = x1 ^ x0
        x0 = x0 + jnp.uint32(keys[(i + 1) % 3])
        x1 = x1 + jnp.uint32((keys[(i + 2) % 3] + i + 1) & 0xFFFFFFFF)
    return x0 ^ x1


def _gumbel_from_bits(bits):
    fb = (bits >> jnp.uint32(9)) | jnp.uint32(0x3F800000)
    floats = jax.lax.bitcast_convert_type(fb, jnp.float32) - jnp.float32(1.0)
    u = jnp.maximum(jnp.float32(_TINY), floats)
    return -jnp.log(-jnp.log(u))


_KD1 = (1832780943, 270669613)
_KD2 = (64467757, 2916123636)


def _body(xr_ref, xf_ref, nr_ref, nf_ref, g1_ref,
          loss_ref, act_ref, acc_ref):
    t = pl.program_id(0)
    row0 = t * _TM

    xr = xr_ref[...]
    xf = xf_ref[...]
    r = jax.lax.dot_general(
        xr, xf, (((1,), (1,)), ((), ())),
        preferred_element_type=jnp.float32)
    d = nr_ref[...] + nf_ref[...] - 2.0 * r
    d = jnp.maximum(d, 1e-08)

    rows = row0 + jax.lax.broadcasted_iota(jnp.int32, (_TM, _B), 0)
    cols = jax.lax.broadcasted_iota(jnp.int32, (_TM, _B), 1)
    in_class = (rows // 16) == (cols // 16)

    dist_pos_full = jnp.where(in_class & (rows != cols), d, 0.0)
    dist_neg = jnp.where(in_class, 1e25, d)

    maxp = jnp.max(dist_pos_full, axis=1, keepdims=True)
    minn = jnp.min(dist_neg, axis=1, keepdims=True)
    acc_flags = (jnp.sqrt(maxp) < jnp.sqrt(minn)).astype(jnp.float32)

    r_in = jax.lax.dot_general(
        xr, xr, (((1,), (1,)), ((), ())),
        preferred_element_type=jnp.float32)
    d_in = nr_ref[...] + nf_ref[:, pl.ds(row0, _TM)] - 2.0 * r_in
    d_in = jnp.maximum(d_in, 1e-08)
    li = jax.lax.broadcasted_iota(jnp.int32, (_TM, _TM), 0)
    lj = jax.lax.broadcasted_iota(jnp.int32, (_TM, _TM), 1)
    pos_ok = ((li // 16) == (lj // 16)) & (li != lj)
    dp_in = jnp.where(pos_ok, d_in, 0.0)
    plog = jnp.where(dp_in > 0.0, jnp.log(jnp.maximum(dp_in, 1e-30)), _NEG_INF)
    g1t = jnp.tile(g1_ref[...], (1, _TM // 16))
    pv = plog + g1t
    m1 = jnp.max(pv, axis=1, keepdims=True)
    idx1 = jnp.min(jnp.where(pv == m1, lj, _B), axis=1, keepdims=True)
    pscore = jnp.sum(jnp.where(lj == idx1, dp_in, 0.0), axis=1, keepdims=True)

    lflat = (jax.lax.broadcasted_iota(jnp.uint32, (_TM, _B), 0) << 12) \
        | jax.lax.broadcasted_iota(jnp.uint32, (_TM, _B), 1)
    g2 = _gumbel_from_bits(
        _threefry_bits(_KD2[0], _KD2[1], lflat,
                       jnp.uint32(t) * jnp.uint32(_TM * _B)))
    candm = dist_neg < (pscore + 0.0001)
    nv = jnp.where(candm, -dist_neg, _NEG_INF) + g2
    m2 = jnp.max(nv, axis=1, keepdims=True)
    idx2 = jnp.min(jnp.where(nv == m2, cols, _B), axis=1, keepdims=True)
    has = m2 > _NEG_INF
    negid = jnp.where(has, idx2, 0)
    sneg = jnp.sum(jnp.where(cols == negid, dist_neg, 0.0), axis=1, keepdims=True)

    score_pos = jnp.sqrt(pscore)
    score_neg = jnp.sqrt(sneg)
    diff = (0.0001 + score_pos) - score_neg
    loss_ref[...] = jnp.logaddexp(diff, 0.0)

    @pl.when(t == 0)
    def _init():
        act_ref[...] = jnp.zeros((1, 1), jnp.float32)
        acc_ref[...] = jnp.zeros((1, 1), jnp.float32)

    act_ref[...] += jnp.sum((diff > 0.0).astype(jnp.float32), keepdims=True)
    acc_ref[...] += jnp.sum(acc_flags, keepdims=True)


@functools.partial(jax.jit, static_argnames=("interpret",))
def _run(features, interpret=False):
    feat_norm = jnp.sum(features * features, axis=1, keepdims=True)
    i = jnp.arange(_B, dtype=jnp.uint32)[:, None]
    kk = jnp.arange(16, dtype=jnp.uint32)[None, :]
    flat = i * jnp.uint32(_B) + (i // jnp.uint32(16)) * jnp.uint32(16) + kk
    g1d = _gumbel_from_bits(_threefry_bits(_KD1[0], _KD1[1], flat))

    loss, act, acc = pl.pallas_call(
        _body,
        grid=(_B // _TM,),
        in_specs=[
            pl.BlockSpec((_TM, _D), lambda t: (t, 0)),
            pl.BlockSpec((_B, _D), lambda t: (0, 0)),
            pl.BlockSpec((_TM, 1), lambda t: (t, 0)),
            pl.BlockSpec((1, _B), lambda t: (0, 0)),
            pl.BlockSpec((_TM, 16), lambda t: (t, 0)),
        ],
        out_specs=[
            pl.BlockSpec((_TM, 1), lambda t: (t, 0)),
            pl.BlockSpec((1, 1), lambda t: (0, 0)),
            pl.BlockSpec((1, 1), lambda t: (0, 0)),
        ],
        out_shape=[
            jax.ShapeDtypeStruct((_B, 1), jnp.float32),
            jax.ShapeDtypeStruct((1, 1), jnp.float32),
            jax.ShapeDtypeStruct((1, 1), jnp.float32),
        ],
        interpret=interpret,
    )(features, features, feat_norm, feat_norm.T, g1d)

    avg_active = act[0, 0] / _B
    accuracy = 100.0 * acc[0, 0] / _B
    return loss, avg_active, accuracy


def kernel(features):
    return _run(features)


# speedup vs baseline: 1.2971x; 1.0530x over previous
import functools

import jax
import jax.numpy as jnp
from jax.experimental import pallas as pl

_B = 4096
_D = 64
_TM = 256
_NEG_INF = float("-inf")
_TINY = float(jnp.finfo(jnp.float32).tiny)


def _threefry_bits(kd0, kd1, idx, offset=0):
    ks2 = (0x1BD11BDA ^ kd0 ^ kd1) & 0xFFFFFFFF
    rots = ((13, 15, 26, 6), (17, 29, 16, 24))
    keys = (kd0, kd1, ks2)
    x0 = jnp.full_like(idx, jnp.uint32(kd0))
    x1 = idx + (jnp.uint32(kd1) + jnp.uint32(offset))
    for i in range(5):
        for rot in rots[i % 2]:
            x0 = x0 + x1
            x1 = (x1 << rot) | (x1 >> (32 - rot))
            x1 = x1 ^ x0
        x0 = x0 + jnp.uint32(keys[(i + 1) % 3])
        x1 = x1 + jnp.uint32((keys[(i + 2) % 3] + i + 1) & 0xFFFFFFFF)
    return x0 ^ x1


def _gumbel_from_bits(bits):
    fb = (bits >> jnp.uint32(9)) | jnp.uint32(0x3F800000)
    floats = jax.lax.bitcast_convert_type(fb, jnp.float32) - jnp.float32(1.0)
    u = jnp.maximum(jnp.float32(_TINY), floats)
    return -jnp.log(-jnp.log(u))


_KD1 = (1832780943, 270669613)
_KD2 = (64467757, 2916123636)


def _body(xr_ref, xf_ref, nr_ref, nf_ref, g1_ref,
          loss_ref, act_ref, acc_ref):
    t = pl.program_id(0)
    row0 = t * _TM

    xr = xr_ref[...]
    xf = xf_ref[...]
    r = jax.lax.dot_general(
        xr, xf, (((1,), (1,)), ((), ())),
        preferred_element_type=jnp.float32)
    d = nr_ref[...] + nf_ref[...] - 2.0 * r
    d = jnp.maximum(d, 1e-08)

    rows = row0 + jax.lax.broadcasted_iota(jnp.int32, (_TM, _B), 0)
    cols = jax.lax.broadcasted_iota(jnp.int32, (_TM, _B), 1)
    in_class = (rows // 16) == (cols // 16)

    dist_neg = jnp.where(in_class, 1e25, d)
    minn = jnp.min(dist_neg, axis=1, keepdims=True)

    r_in = jax.lax.dot_general(
        xr, xr, (((1,), (1,)), ((), ())),
        preferred_element_type=jnp.float32)
    d_in = nr_ref[...] + nf_ref[:, pl.ds(row0, _TM)] - 2.0 * r_in
    d_in = jnp.maximum(d_in, 1e-08)
    li = jax.lax.broadcasted_iota(jnp.int32, (_TM, _TM), 0)
    lj = jax.lax.broadcasted_iota(jnp.int32, (_TM, _TM), 1)
    pos_ok = ((li // 16) == (lj // 16)) & (li != lj)
    dp_in = jnp.where(pos_ok, d_in, 0.0)
    maxp = jnp.max(dp_in, axis=1, keepdims=True)
    acc_flags = (jnp.sqrt(maxp) < jnp.sqrt(minn)).astype(jnp.float32)
    plog = jnp.where(dp_in > 0.0, jnp.log(jnp.maximum(dp_in, 1e-30)), _NEG_INF)
    g1t = jnp.tile(g1_ref[...], (1, _TM // 16))
    pv = plog + g1t
    m1 = jnp.max(pv, axis=1, keepdims=True)
    idx1 = jnp.min(jnp.where(pv == m1, lj, _B), axis=1, keepdims=True)
    pscore = jnp.sum(jnp.where(lj == idx1, dp_in, 0.0), axis=1, keepdims=True)

    lflat = (jax.lax.broadcasted_iota(jnp.uint32, (_TM, _B), 0) << 12) \
        | jax.lax.broadcasted_iota(jnp.uint32, (_TM, _B), 1)
    g2 = _gumbel_from_bits(
        _threefry_bits(_KD2[0], _KD2[1], lflat,
                       jnp.uint32(t) * jnp.uint32(_TM * _B)))
    candm = dist_neg < (pscore + 0.0001)
    nv = jnp.where(candm, -dist_neg, _NEG_INF) + g2
    m2 = jnp.max(nv, axis=1, keepdims=True)
    idx2 = jnp.min(jnp.where(nv == m2, cols, _B), axis=1, keepdims=True)
    has = m2 > _NEG_INF
    negid = jnp.where(has, idx2, 0)
    sneg = jnp.sum(jnp.where(cols == negid, dist_neg, 0.0), axis=1, keepdims=True)

    score_pos = jnp.sqrt(pscore)
    score_neg = jnp.sqrt(sneg)
    diff = (0.0001 + score_pos) - score_neg
    loss_ref[...] = jnp.logaddexp(diff, 0.0)

    @pl.when(t == 0)
    def _init():
        act_ref[...] = jnp.zeros((1, 1), jnp.float32)
        acc_ref[...] = jnp.zeros((1, 1), jnp.float32)

    act_ref[...] += jnp.sum((diff > 0.0).astype(jnp.float32), keepdims=True)
    acc_ref[...] += jnp.sum(acc_flags, keepdims=True)


@functools.partial(jax.jit, static_argnames=("interpret",))
def _run(features, interpret=False):
    feat_norm = jnp.sum(features * features, axis=1, keepdims=True)
    i = jnp.arange(_B, dtype=jnp.uint32)[:, None]
    kk = jnp.arange(16, dtype=jnp.uint32)[None, :]
    flat = i * jnp.uint32(_B) + (i // jnp.uint32(16)) * jnp.uint32(16) + kk
    g1d = _gumbel_from_bits(_threefry_bits(_KD1[0], _KD1[1], flat))

    loss, act, acc = pl.pallas_call(
        _body,
        grid=(_B // _TM,),
        in_specs=[
            pl.BlockSpec((_TM, _D), lambda t: (t, 0)),
            pl.BlockSpec((_B, _D), lambda t: (0, 0)),
            pl.BlockSpec((_TM, 1), lambda t: (t, 0)),
            pl.BlockSpec((1, _B), lambda t: (0, 0)),
            pl.BlockSpec((_TM, 16), lambda t: (t, 0)),
        ],
        out_specs=[
            pl.BlockSpec((_TM, 1), lambda t: (t, 0)),
            pl.BlockSpec((1, 1), lambda t: (0, 0)),
            pl.BlockSpec((1, 1), lambda t: (0, 0)),
        ],
        out_shape=[
            jax.ShapeDtypeStruct((_B, 1), jnp.float32),
            jax.ShapeDtypeStruct((1, 1), jnp.float32),
            jax.ShapeDtypeStruct((1, 1), jnp.float32),
        ],
        interpret=interpret,
    )(features, features, feat_norm, feat_norm.T, g1d)

    avg_active = act[0, 0] / _B
    accuracy = 100.0 * acc[0, 0] / _B
    return loss, avg_active, accuracy


def kernel(features):
    return _run(features)


# speedup vs baseline: 1.3125x; 1.0119x over previous
import functools

import jax
import jax.numpy as jnp
from jax.experimental import pallas as pl

_B = 4096
_D = 64
_TM = 256
_NEG_INF = float("-inf")
_TINY = float(jnp.finfo(jnp.float32).tiny)


def _threefry_bits(kd0, kd1, idx, offset=0):
    ks2 = (0x1BD11BDA ^ kd0 ^ kd1) & 0xFFFFFFFF
    rots = ((13, 15, 26, 6), (17, 29, 16, 24))
    keys = (kd0, kd1, ks2)
    x0 = jnp.full_like(idx, jnp.uint32(kd0))
    x1 = idx + (jnp.uint32(kd1) + jnp.uint32(offset))
    for i in range(5):
        for rot in rots[i % 2]:
            x0 = x0 + x1
            x1 = (x1 << rot) | (x1 >> (32 - rot))
            x1 = x1 ^ x0
        x0 = x0 + jnp.uint32(keys[(i + 1) % 3])
        x1 = x1 + jnp.uint32((keys[(i + 2) % 3] + i + 1) & 0xFFFFFFFF)
    return x0 ^ x1


def _gumbel_from_bits(bits):
    fb = (bits >> jnp.uint32(9)) | jnp.uint32(0x3F800000)
    floats = jax.lax.bitcast_convert_type(fb, jnp.float32) - jnp.float32(1.0)
    u = jnp.maximum(jnp.float32(_TINY), floats)
    return -jnp.log(-jnp.log(u))


_KD1 = (1832780943, 270669613)
_KD2 = (64467757, 2916123636)


def _body(xr_ref, xf_ref, nr_ref, nf_ref, g1_ref,
          loss_ref, act_ref, acc_ref):
    t = pl.program_id(0)
    row0 = t * _TM

    xr = xr_ref[...]
    xf = xf_ref[...]
    r = jax.lax.dot_general(
        xr, xf, (((1,), (1,)), ((), ())),
        preferred_element_type=jnp.float32)
    d = nr_ref[...] + nf_ref[...] - 2.0 * r
    d = jnp.maximum(d, 1e-08)

    rows = row0 + jax.lax.broadcasted_iota(jnp.int32, (_TM, _B), 0)
    cols = jax.lax.broadcasted_iota(jnp.int32, (_TM, _B), 1)
    in_class = (rows // 16) == (cols // 16)

    dist_neg = jnp.where(in_class, 1e25, d)
    minn = jnp.min(dist_neg, axis=1, keepdims=True)

    r_in = jax.lax.dot_general(
        xr, xr, (((1,), (1,)), ((), ())),
        preferred_element_type=jnp.float32)
    d_in = nr_ref[...] + nf_ref[:, pl.ds(row0, _TM)] - 2.0 * r_in
    d_in = jnp.maximum(d_in, 1e-08)
    li = jax.lax.broadcasted_iota(jnp.int32, (_TM, _TM), 0)
    lj = jax.lax.broadcasted_iota(jnp.int32, (_TM, _TM), 1)
    pos_ok = ((li // 16) == (lj // 16)) & (li != lj)
    dp_in = jnp.where(pos_ok, d_in, 0.0)
    maxp = jnp.max(dp_in, axis=1, keepdims=True)
    acc_flags = (jnp.sqrt(maxp) < jnp.sqrt(minn)).astype(jnp.float32)
    plog = jnp.where(dp_in > 0.0, jnp.log(jnp.maximum(dp_in, 1e-30)), _NEG_INF)
    g1t = jnp.tile(g1_ref[...], (1, _TM // 16))
    pv = plog + g1t
    idx1 = jnp.argmax(pv, axis=1)[:, None].astype(jnp.int32)
    pscore = jnp.sum(jnp.where(lj == idx1, dp_in, 0.0), axis=1, keepdims=True)

    lflat = (jax.lax.broadcasted_iota(jnp.uint32, (_TM, _B), 0) << 12) \
        | jax.lax.broadcasted_iota(jnp.uint32, (_TM, _B), 1)
    g2 = _gumbel_from_bits(
        _threefry_bits(_KD2[0], _KD2[1], lflat,
                       jnp.uint32(t) * jnp.uint32(_TM * _B)))
    thresh = pscore + 0.0001
    candm = dist_neg < thresh
    nv = jnp.where(candm, -dist_neg, _NEG_INF) + g2
    idx2 = jnp.argmax(nv, axis=1)[:, None].astype(jnp.int32)
    has = minn < thresh
    negid = jnp.where(has, idx2, 0)
    sneg = jnp.sum(jnp.where(cols == negid, dist_neg, 0.0), axis=1, keepdims=True)

    score_pos = jnp.sqrt(pscore)
    score_neg = jnp.sqrt(sneg)
    diff = (0.0001 + score_pos) - score_neg
    loss_ref[...] = jnp.logaddexp(diff, 0.0)

    @pl.when(t == 0)
    def _init():
        act_ref[...] = jnp.zeros((1, 1), jnp.float32)
        acc_ref[...] = jnp.zeros((1, 1), jnp.float32)

    act_ref[...] += jnp.sum((diff > 0.0).astype(jnp.float32), keepdims=True)
    acc_ref[...] += jnp.sum(acc_flags, keepdims=True)


@functools.partial(jax.jit, static_argnames=("interpret",))
def _run(features, interpret=False):
    feat_norm = jnp.sum(features * features, axis=1, keepdims=True)
    i = jnp.arange(_B, dtype=jnp.uint32)[:, None]
    kk = jnp.arange(16, dtype=jnp.uint32)[None, :]
    flat = i * jnp.uint32(_B) + (i // jnp.uint32(16)) * jnp.uint32(16) + kk
    g1d = _gumbel_from_bits(_threefry_bits(_KD1[0], _KD1[1], flat))

    loss, act, acc = pl.pallas_call(
        _body,
        grid=(_B // _TM,),
        in_specs=[
            pl.BlockSpec((_TM, _D), lambda t: (t, 0)),
            pl.BlockSpec((_B, _D), lambda t: (0, 0)),
            pl.BlockSpec((_TM, 1), lambda t: (t, 0)),
            pl.BlockSpec((1, _B), lambda t: (0, 0)),
            pl.BlockSpec((_TM, 16), lambda t: (t, 0)),
        ],
        out_specs=[
            pl.BlockSpec((_TM, 1), lambda t: (t, 0)),
            pl.BlockSpec((1, 1), lambda t: (0, 0)),
            pl.BlockSpec((1, 1), lambda t: (0, 0)),
        ],
        out_shape=[
            jax.ShapeDtypeStruct((_B, 1), jnp.float32),
            jax.ShapeDtypeStruct((1, 1), jnp.float32),
            jax.ShapeDtypeStruct((1, 1), jnp.float32),
        ],
        interpret=interpret,
    )(features, features, feat_norm, feat_norm.T, g1d)

    avg_active = act[0, 0] / _B
    accuracy = 100.0 * acc[0, 0] / _B
    return loss, avg_active, accuracy


def kernel(features):
    return _run(features)


# speedup vs baseline: 1.3249x; 1.0094x over previous
import functools

import jax
import jax.numpy as jnp
from jax.experimental import pallas as pl

_B = 4096
_D = 64
_TM = 256
_NEG_INF = float("-inf")
_TINY = float(jnp.finfo(jnp.float32).tiny)


def _threefry_bits(kd0, kd1, idx, offset=0):
    ks2 = (0x1BD11BDA ^ kd0 ^ kd1) & 0xFFFFFFFF
    rots = ((13, 15, 26, 6), (17, 29, 16, 24))
    keys = (kd0, kd1, ks2)
    x0 = jnp.full_like(idx, jnp.uint32(kd0))
    x1 = idx + (jnp.uint32(kd1) + jnp.uint32(offset))
    for i in range(5):
        for rot in rots[i % 2]:
            x0 = x0 + x1
            x1 = (x1 << rot) | (x1 >> (32 - rot))
            x1 = x1 ^ x0
        x0 = x0 + jnp.uint32(keys[(i + 1) % 3])
        x1 = x1 + jnp.uint32((keys[(i + 2) % 3] + i + 1) & 0xFFFFFFFF)
    return x0 ^ x1


def _gumbel_from_bits(bits):
    fb = (bits >> jnp.uint32(9)) | jnp.uint32(0x3F800000)
    floats = jax.lax.bitcast_convert_type(fb, jnp.float32) - jnp.float32(1.0)
    u = jnp.maximum(jnp.float32(_TINY), floats)
    return -jnp.log(-jnp.log(u))


_KD1 = (1832780943, 270669613)
_KD2 = (64467757, 2916123636)


def _body(xr_ref, xf_ref, nr_ref, nf_ref, g1_ref,
          loss_ref, act_ref, acc_ref):
    t = pl.program_id(0)
    row0 = t * _TM

    xr = xr_ref[...]
    xf = xf_ref[...]
    r = jax.lax.dot_general(
        xr, xf, (((1,), (1,)), ((), ())),
        preferred_element_type=jnp.float32)
    d = nr_ref[...] + nf_ref[...] - 2.0 * r
    d = jnp.maximum(d, 1e-08)

    rows = row0 + jax.lax.broadcasted_iota(jnp.int32, (_TM, _B), 0)
    cols = jax.lax.broadcasted_iota(jnp.int32, (_TM, _B), 1)
    in_class = (rows // 16) == (cols // 16)

    dist_neg = jnp.where(in_class, 1e25, d)
    minn = jnp.min(dist_neg, axis=1, keepdims=True)

    r_in = jax.lax.dot_general(
        xr, xr, (((1,), (1,)), ((), ())),
        preferred_element_type=jnp.float32)
    d_in = nr_ref[...] + nf_ref[:, pl.ds(row0, _TM)] - 2.0 * r_in
    d_in = jnp.maximum(d_in, 1e-08)
    li = jax.lax.broadcasted_iota(jnp.int32, (_TM, _TM), 0)
    lj = jax.lax.broadcasted_iota(jnp.int32, (_TM, _TM), 1)
    pos_ok = ((li // 16) == (lj // 16)) & (li != lj)
    dp_in = jnp.where(pos_ok, d_in, 0.0)
    maxp = jnp.max(dp_in, axis=1, keepdims=True)
    acc_flags = (jnp.sqrt(maxp) < jnp.sqrt(minn)).astype(jnp.float32)
    plog = jnp.where(dp_in > 0.0, jnp.log(jnp.maximum(dp_in, 1e-30)), _NEG_INF)
    g1t = jnp.tile(g1_ref[...], (1, _TM // 16))
    pv = plog + g1t
    idx1 = jnp.argmax(pv, axis=1)[:, None].astype(jnp.int32)
    pscore = jnp.sum(jnp.where(lj == idx1, dp_in, 0.0), axis=1, keepdims=True)

    lflat = (jax.lax.broadcasted_iota(jnp.uint32, (_TM, _B), 0) << 12) \
        | jax.lax.broadcasted_iota(jnp.uint32, (_TM, _B), 1)
    g2 = _gumbel_from_bits(
        _threefry_bits(_KD2[0], _KD2[1], lflat,
                       jnp.uint32(t) * jnp.uint32(_TM * _B)))
    thresh = pscore + 0.0001
    nv = jnp.where(dist_neg < thresh, g2 - dist_neg, _NEG_INF)
    idx2 = jnp.argmax(nv, axis=1)[:, None].astype(jnp.int32)
    has = minn < thresh
    negid = jnp.where(has, idx2, 0)
    sneg = jnp.sum(jnp.where(cols == negid, dist_neg, 0.0), axis=1, keepdims=True)

    score_pos = jnp.sqrt(pscore)
    score_neg = jnp.sqrt(sneg)
    diff = (0.0001 + score_pos) - score_neg
    loss_ref[...] = jnp.logaddexp(diff, 0.0)

    @pl.when(t == 0)
    def _init():
        act_ref[...] = jnp.zeros((1, 1), jnp.float32)
        acc_ref[...] = jnp.zeros((1, 1), jnp.float32)

    act_ref[...] += jnp.sum((diff > 0.0).astype(jnp.float32), keepdims=True)
    acc_ref[...] += jnp.sum(acc_flags, keepdims=True)


@functools.partial(jax.jit, static_argnames=("interpret",))
def _run(features, interpret=False):
    feat_norm = jnp.sum(features * features, axis=1, keepdims=True)
    i = jnp.arange(_B, dtype=jnp.uint32)[:, None]
    kk = jnp.arange(16, dtype=jnp.uint32)[None, :]
    flat = i * jnp.uint32(_B) + (i // jnp.uint32(16)) * jnp.uint32(16) + kk
    g1d = _gumbel_from_bits(_threefry_bits(_KD1[0], _KD1[1], flat))

    loss, act, acc = pl.pallas_call(
        _body,
        grid=(_B // _TM,),
        in_specs=[
            pl.BlockSpec((_TM, _D), lambda t: (t, 0)),
            pl.BlockSpec((_B, _D), lambda t: (0, 0)),
            pl.BlockSpec((_TM, 1), lambda t: (t, 0)),
            pl.BlockSpec((1, _B), lambda t: (0, 0)),
            pl.BlockSpec((_TM, 16), lambda t: (t, 0)),
        ],
        out_specs=[
            pl.BlockSpec((_TM, 1), lambda t: (t, 0)),
            pl.BlockSpec((1, 1), lambda t: (0, 0)),
            pl.BlockSpec((1, 1), lambda t: (0, 0)),
        ],
        out_shape=[
            jax.ShapeDtypeStruct((_B, 1), jnp.float32),
            jax.ShapeDtypeStruct((1, 1), jnp.float32),
            jax.ShapeDtypeStruct((1, 1), jnp.float32),
        ],
        interpret=interpret,
    )(features, features, feat_norm, feat_norm.T, g1d)

    avg_active = act[0, 0] / _B
    accuracy = 100.0 * acc[0, 0] / _B
    return loss, avg_active, accuracy


def kernel(features):
    return _run(features)
